# Initial kernel scaffold; baseline (speedup 1.0000x reference)
import jax
import jax.numpy as jnp
from jax.experimental import pallas as pl


def kernel(H, Xe, id_Xe, batch_idx, params):
    raise NotImplementedError("write your pallas kernel here")


# speedup vs baseline: 3.2238x; 3.2238x over previous
import functools

import jax
import jax.numpy as jnp
from jax import lax
from jax.experimental import pallas as pl
from jax.experimental.pallas import tpu as pltpu
from jax.experimental.pallas import tpu_sc as plsc

N = 10000
E = 320000
DF = 128
DE = 16
HD = 128
G = 64

NP = 10240
NC, NS, L = 2, 16, 16
NW = NC * NS
EPT = E // NW
CHUNK = 80
GRP = 25
NGRP = EPT // (GRP * CHUNK)
RPT = NP // NS
F32 = jnp.float32


def _sc_edge_body(z_hbm, c_hbm, src_hbm, dst_hbm, out_hbm,
                  sidx, didx, zrow, crow, agg_sh, sem):
    c = lax.axis_index("c")
    s = lax.axis_index("s")
    tile = c * NS + s

    def zset(i, carry):
        for k in range(HD // L):
            zrow[i, pl.ds(k * L, L)] = jnp.zeros((L,), F32)
        return carry
    lax.fori_loop(0, CHUNK, zset, 0)
    for q in range(RPT // CHUNK):
        pltpu.sync_copy(zrow, agg_sh.at[pl.ds(s * RPT + q * CHUNK, CHUNK)])
    plsc.subcore_barrier()

    def group_body(g, carry):
        pltpu.sync_copy(src_hbm.at[tile, g], sidx)
        pltpu.sync_copy(dst_hbm.at[tile, g], didx)

        def chunk_body(j, cc):
            base = tile * EPT + (g * GRP + j) * CHUNK
            pltpu.async_copy(z_hbm.at[sidx.at[j]], zrow, sem).wait()
            pltpu.sync_copy(c_hbm.at[pl.ds(base, CHUNK)], crow)

            def erow(e, c2):
                for k in range(HD // L):
                    sl = pl.ds(k * L, L)
                    zrow[e, sl] = jnp.maximum(zrow[e, sl] + crow[e, sl], 0.0)
                return c2
            lax.fori_loop(0, CHUNK, erow, 0)
            pltpu.sync_copy(zrow, agg_sh.at[didx.at[j]], add=True)
            return cc
        lax.fori_loop(0, GRP, chunk_body, 0)
        return carry
    lax.fori_loop(0, NGRP, group_body, 0)
    plsc.subcore_barrier()

    for q in range(RPT // CHUNK):
        r0 = s * RPT + q * CHUNK
        pltpu.sync_copy(agg_sh.at[pl.ds(r0, CHUNK)], zrow)
        pltpu.sync_copy(zrow, out_hbm.at[c, pl.ds(r0, CHUNK)])


_sc_edge_pass = pl.kernel(
    _sc_edge_body,
    out_type=jax.ShapeDtypeStruct((NC, NP, HD), F32),
    mesh=plsc.VectorSubcoreMesh(core_axis_name="c", subcore_axis_name="s",
                                num_cores=NC, num_subcores=NS),
    scratch_types=[
        pltpu.VMEM((GRP, CHUNK), jnp.int32),
        pltpu.VMEM((GRP, CHUNK), jnp.int32),
        pltpu.VMEM((CHUNK, HD), F32),
        pltpu.VMEM((CHUNK, HD), F32),
        pltpu.VMEM_SHARED((NP, HD), F32),
        pltpu.SemaphoreType.DMA,
    ],
)


def _mm_bias_body(x_ref, w_ref, b_ref, o_ref):
    o_ref[...] = (jnp.dot(x_ref[...], w_ref[...], preferred_element_type=F32)
                  + b_ref[...])


def _mm_bias(x, w, b, bm):
    m, k = x.shape
    hd = w.shape[1]
    return pl.pallas_call(
        _mm_bias_body,
        grid=(m // bm,),
        in_specs=[
            pl.BlockSpec((bm, k), lambda i: (i, 0)),
            pl.BlockSpec((k, hd), lambda i: (0, 0)),
            pl.BlockSpec((1, hd), lambda i: (0, 0)),
        ],
        out_specs=pl.BlockSpec((bm, hd), lambda i: (i, 0)),
        out_shape=jax.ShapeDtypeStruct((m, hd), F32),
    )(x, w, b.reshape(1, hd))


BM = 2048


def _onehot(b_ref):
    ids = jax.lax.broadcasted_iota(jnp.int32, (BM, G), 1)
    return (b_ref[...] == ids).astype(F32)


def _update_pool_body(p0, p1, y, wua, wub, bu, b_ref, o_y, o_pool):
    agg = p0[...] + p1[...]
    yn = jnp.maximum(
        jnp.dot(agg, wua[...], preferred_element_type=F32)
        + jnp.dot(y[...], wub[...], preferred_element_type=F32)
        + bu[...], 0.0)
    o_y[...] = yn

    @pl.when(pl.program_id(0) == 0)
    def _():
        o_pool[...] = jnp.zeros_like(o_pool)
    oh = _onehot(b_ref)
    o_pool[...] += jax.lax.dot_general(
        oh, yn, (((0,), (0,)), ((), ())), preferred_element_type=F32)


def _update_pool(p0, p1, y, wua, wub, bu, bidx):
    return pl.pallas_call(
        _update_pool_body,
        grid=(NP // BM,),
        in_specs=[
            pl.BlockSpec((BM, HD), lambda i: (i, 0)),
            pl.BlockSpec((BM, HD), lambda i: (i, 0)),
            pl.BlockSpec((BM, HD), lambda i: (i, 0)),
            pl.BlockSpec((HD, HD), lambda i: (0, 0)),
            pl.BlockSpec((HD, HD), lambda i: (0, 0)),
            pl.BlockSpec((1, HD), lambda i: (0, 0)),
            pl.BlockSpec((BM, 1), lambda i: (i, 0)),
        ],
        out_specs=[
            pl.BlockSpec((BM, HD), lambda i: (i, 0)),
            pl.BlockSpec((G, HD), lambda i: (0, 0)),
        ],
        out_shape=[
            jax.ShapeDtypeStruct((NP, HD), F32),
            jax.ShapeDtypeStruct((G, HD), F32),
        ],
    )(p0, p1, y, wua, wub, bu.reshape(1, HD), bidx)


def _vn_z_body(y, pool, wv, bv, b_ref, wma, o_y2, o_z):
    v = jnp.maximum(
        jnp.dot(pool[...], wv[...], preferred_element_type=F32) + bv[...], 0.0)
    oh = _onehot(b_ref)
    y2 = y[...] + jnp.dot(oh, v, preferred_element_type=F32)
    o_y2[...] = y2
    o_z[...] = jnp.dot(y2, wma[...], preferred_element_type=F32)


def _vn_z(y, pool, wv, bv, bidx, wma):
    return pl.pallas_call(
        _vn_z_body,
        grid=(NP // BM,),
        in_specs=[
            pl.BlockSpec((BM, HD), lambda i: (i, 0)),
            pl.BlockSpec((G, HD), lambda i: (0, 0)),
            pl.BlockSpec((HD, HD), lambda i: (0, 0)),
            pl.BlockSpec((1, HD), lambda i: (0, 0)),
            pl.BlockSpec((BM, 1), lambda i: (i, 0)),
            pl.BlockSpec((HD, HD), lambda i: (0, 0)),
        ],
        out_specs=[
            pl.BlockSpec((BM, HD), lambda i: (i, 0)),
            pl.BlockSpec((BM, HD), lambda i: (i, 0)),
        ],
        out_shape=[
            jax.ShapeDtypeStruct((NP, HD), F32),
            jax.ShapeDtypeStruct((NP, HD), F32),
        ],
    )(y, pool, wv, bv.reshape(1, HD), bidx, wma)


def _head_body(pool, wout, bout, o_ref):
    o_ref[...] = (jnp.dot(pool[...], wout[...], preferred_element_type=F32)
                  + bout[...])


def _head(pool, wout, bout):
    return pl.pallas_call(
        _head_body,
        grid=(1,),
        in_specs=[
            pl.BlockSpec((G, HD), lambda i: (0, 0)),
            pl.BlockSpec((HD, 1), lambda i: (0, 0)),
            pl.BlockSpec((1, 1), lambda i: (0, 0)),
        ],
        out_specs=pl.BlockSpec((G, 1), lambda i: (0, 0)),
        out_shape=jax.ShapeDtypeStruct((G, 1), F32),
    )(pool, wout, bout.reshape(1, 1))


def kernel(H, Xe, id_Xe, batch_idx, params):
    src = id_Xe[0].reshape(NW, NGRP, GRP, CHUNK)
    dst = id_Xe[1].reshape(NW, NGRP, GRP, CHUNK)
    Hp = jnp.pad(H, ((0, NP - N), (0, 0)))
    bidx = jnp.pad(batch_idx, (0, NP - N), constant_values=G).reshape(NP, 1)

    p = params
    Wm = [p['Wm0'], p['Wm1'], p['Wm2']]
    bm = [p['bm0'], p['bm1'], p['bm2']]
    Wu = [p['Wu0'], p['Wu1'], p['Wu2']]
    bu = [p['bu0'], p['bu1'], p['bu2']]
    Wv = [p['Wv0'], p['Wv1']]
    bv = [p['bv0'], p['bv1']]

    C = [_mm_bias(Xe, Wm[l][DF:], bm[l], 3200) for l in range(3)]

    y = Hp
    Z = _mm_bias(Hp, Wm[0][:DF], jnp.zeros((HD,), F32), BM)
    pool = None
    for l in range(3):
        P = _sc_edge_pass(Z, C[l], src, dst)
        y, pool = _update_pool(P[0], P[1], y, Wu[l][:HD], Wu[l][HD:],
                               bu[l], bidx)
        if l < 2:
            y, Z = _vn_z(y, pool, Wv[l], bv[l], bidx, Wm[l + 1][:DF])

    return _head(pool, p['Wout'], p['bout'])
